# Initial kernel scaffold; baseline (speedup 1.0000x reference)
import jax
import jax.numpy as jnp
from jax import lax
from jax.experimental import pallas as pl
from jax.experimental.pallas import tpu as pltpu
from jax.experimental.pallas import tpu_sc as plsc

N = 10000
E = 320000
D = 128
NC, NS = 2, 16
NW = NC * NS
EPT = E // NW
CH = 80
NCHUNK = EPT // CH
DEG_PAD = 10240
DEG_STR = DEG_PAD // NS
ROWS_PT = N // NS

_GDN = lax.GatherDimensionNumbers(
    offset_dims=(), collapsed_slice_dims=(0,), start_index_map=(0,))


def _bcast16(v, lane):
  idx = jnp.full((16,), lane, jnp.int32)
  return lax.gather(v, idx[:, None], _GDN, (1,),
                    mode=lax.GatherScatterMode.PROMISE_IN_BOUNDS)


def _zero_vmem_1d(ref, n):
  def body(i, _):
    ref[pl.ds(i * 16, 16)] = jnp.zeros((16,), jnp.float32)
    return 0
  lax.fori_loop(0, n // 16, body, 0)


def _zero_vmem_rows(ref, rows):
  def body(i, _):
    for j in range(D // 16):
      ref[i, pl.ds(j * 16, 16)] = jnp.zeros((16,), jnp.float32)
    return 0
  lax.fori_loop(0, rows, body, 0)


def _deg_body(dst_hbm, ew_hbm, out_hbm, idx_v, ew_v, z_v, acc_sh):
  c = lax.axis_index("c")
  s = lax.axis_index("s")
  base = (c * NS + s) * EPT

  _zero_vmem_1d(z_v, DEG_STR)
  pltpu.sync_copy(z_v, acc_sh.at[pl.ds(s * DEG_STR, DEG_STR)])
  plsc.subcore_barrier()

  def chunk(i, _):
    off = base + i * CH
    pltpu.sync_copy(dst_hbm.at[pl.ds(off, CH)], idx_v)
    pltpu.sync_copy(ew_hbm.at[pl.ds(off, CH)], ew_v)
    pltpu.sync_copy(ew_v, acc_sh.at[idx_v], add=True)
    return 0
  lax.fori_loop(0, NCHUNK, chunk, 0)

  plsc.subcore_barrier()
  pltpu.sync_copy(acc_sh.at[pl.ds(s * DEG_STR, DEG_STR)],
                  out_hbm.at[c, pl.ds(s * DEG_STR, DEG_STR)])


_deg_call = pl.kernel(
    _deg_body,
    out_type=jax.ShapeDtypeStruct((NC, DEG_PAD), jnp.float32),
    mesh=plsc.VectorSubcoreMesh(core_axis_name="c", subcore_axis_name="s"),
    scratch_types=[
        pltpu.VMEM((CH,), jnp.int32),
        pltpu.VMEM((CH,), jnp.float32),
        pltpu.VMEM((DEG_STR,), jnp.float32),
        pltpu.VMEM_SHARED((DEG_PAD,), jnp.float32),
    ],
)


def _agg_body(hp_hbm, src_hbm, dst_hbm, ew_hbm, out_hbm,
              sidx, didx, ewv, rows, zrows, acc_sh, gsem):
  c = lax.axis_index("c")
  s = lax.axis_index("s")
  base = (c * NS + s) * EPT

  _zero_vmem_rows(zrows, 125)
  for t in range(5):
    pltpu.sync_copy(zrows, acc_sh.at[pl.ds(s * ROWS_PT + t * 125, 125)])
  plsc.subcore_barrier()

  def chunk(i, _):
    off = base + i * CH
    pltpu.sync_copy(src_hbm.at[pl.ds(off, CH)], sidx)
    pltpu.sync_copy(ew_hbm.at[pl.ds(off, CH)], ewv)
    pltpu.sync_copy(dst_hbm.at[pl.ds(off, CH)], didx)
    pltpu.async_copy(hp_hbm.at[sidx], rows, gsem).wait()

    def grp(g, _):
      wv = ewv[pl.ds(g * 16, 16)]
      for e in range(16):
        w = _bcast16(wv, e)
        r = g * 16 + e
        for j in range(D // 16):
          rows[r, pl.ds(j * 16, 16)] = rows[r, pl.ds(j * 16, 16)] * w
      return 0
    lax.fori_loop(0, CH // 16, grp, 0)

    pltpu.sync_copy(rows, acc_sh.at[didx], add=True)
    return 0
  lax.fori_loop(0, NCHUNK, chunk, 0)

  plsc.subcore_barrier()
  pltpu.sync_copy(acc_sh.at[pl.ds(s * ROWS_PT, ROWS_PT)],
                  out_hbm.at[c, pl.ds(s * ROWS_PT, ROWS_PT)])


_agg_call = pl.kernel(
    _agg_body,
    out_type=jax.ShapeDtypeStruct((NC, N, D), jnp.float32),
    mesh=plsc.VectorSubcoreMesh(core_axis_name="c", subcore_axis_name="s"),
    scratch_types=[
        pltpu.VMEM((CH,), jnp.int32),
        pltpu.VMEM((CH,), jnp.int32),
        pltpu.VMEM((CH,), jnp.float32),
        pltpu.VMEM((CH, D), jnp.float32),
        pltpu.VMEM((125, D), jnp.float32),
        pltpu.VMEM_SHARED((N, D), jnp.float32),
        pltpu.SemaphoreType.DMA,
    ],
)


def _leaky(x):
  return jnp.where(x >= 0.0, x, 0.1 * x)


def _bn(x, gamma, beta):
  mu = jnp.mean(x, axis=0, keepdims=True)
  xc = x - mu
  var = jnp.mean(xc * xc, axis=0, keepdims=True)
  return gamma * xc * lax.rsqrt(var + 1e-5) + beta


def _tc1_body(x_ref, d0_ref, d1_ref, g_ref, bt_ref, w_ref, h1p_ref, dinv_ref):
  x = x_ref[...]
  xa = _leaky(_bn(x, g_ref[...], bt_ref[...]))
  deg = d0_ref[...] + d1_ref[...] + 1.0
  dinv = jnp.where(deg > 0.0, lax.rsqrt(deg), 0.0)
  h = jnp.dot(xa, w_ref[...], preferred_element_type=jnp.float32)
  h1p_ref[...] = dinv * h
  dinv_ref[...] = dinv


def _tc2_body(p0_ref, p1_ref, hp_ref, dinv_ref, b1_ref, g_ref, bt_ref, w_ref,
              h2p_ref):
  dinv = dinv_ref[...]
  o1 = dinv * (p0_ref[...] + p1_ref[...] + hp_ref[...]) + b1_ref[...]
  a = _leaky(_bn(_leaky(o1), g_ref[...], bt_ref[...]))
  h2 = jnp.dot(a, w_ref[...], preferred_element_type=jnp.float32)
  h2p_ref[...] = dinv * h2


def _tc3_body(q0_ref, q1_ref, hp_ref, dinv_ref, b2_ref, out_ref):
  o = dinv_ref[...] * (q0_ref[...] + q1_ref[...] + hp_ref[...]) + b2_ref[...]
  out_ref[...] = _leaky(o)


_f32 = jnp.float32
_tc1_call = pl.pallas_call(
    _tc1_body,
    out_shape=(jax.ShapeDtypeStruct((N, D), _f32),
               jax.ShapeDtypeStruct((N, 1), _f32)),
)
_tc2_call = pl.pallas_call(
    _tc2_body,
    out_shape=jax.ShapeDtypeStruct((N, D), _f32),
)
_tc3_call = pl.pallas_call(
    _tc3_body,
    out_shape=jax.ShapeDtypeStruct((N, D), _f32),
)


def kernel(x, edge_index, edge_attr, bn1_gamma, bn1_beta, W1, b1,
           bn2_gamma, bn2_beta, W2, b2):
  src = edge_index[0]
  dst = edge_index[1]
  ew = edge_attr[:, 0]

  deg_parts = _deg_call(dst, ew)
  d0 = deg_parts[0, :N, None]
  d1 = deg_parts[1, :N, None]

  h1p, dinv = _tc1_call(x, d0, d1, bn1_gamma[None, :], bn1_beta[None, :], W1)

  p = _agg_call(h1p, src, dst, ew)
  h2p = _tc2_call(p[0], p[1], h1p, dinv, b1[None, :],
                  bn2_gamma[None, :], bn2_beta[None, :], W2)

  q = _agg_call(h2p, src, dst, ew)
  out = _tc3_call(q[0], q[1], h2p, dinv, b2[None, :])
  return (out, edge_index)


# speedup vs baseline: 8.8042x; 8.8042x over previous
import jax
import jax.numpy as jnp
from jax import lax
from jax.experimental import pallas as pl
from jax.experimental.pallas import tpu as pltpu
from jax.experimental.pallas import tpu_sc as plsc

N = 10000
E = 320000
D = 128
NC, NS = 2, 16
NW = NC * NS
EPT = E // NW
CH = 80
NCHUNK = EPT // CH
APAD = 10240
STR = APAD // NS

_GDN = lax.GatherDimensionNumbers(
    offset_dims=(), collapsed_slice_dims=(0,), start_index_map=(0,))


def _bcast16(v, lane):
  idx = jnp.full((16,), lane, jnp.int32)
  return lax.gather(v, idx[:, None], _GDN, (1,),
                    mode=lax.GatherScatterMode.PROMISE_IN_BOUNDS)


def _zero_vmem_1d(ref, n):
  def body(i, _):
    ref[pl.ds(i * 16, 16)] = jnp.zeros((16,), jnp.float32)
    return 0
  lax.fori_loop(0, n // 16, body, 0)


def _zero_vmem_rows(ref, rows):
  def body(i, _):
    for j in range(D // 16):
      ref[i, pl.ds(j * 16, 16)] = jnp.zeros((16,), jnp.float32)
    return 0
  lax.fori_loop(0, rows, body, 0)


def _deg_body(dst_hbm, ew_hbm, out_hbm, idx_v, ew_v, z_v, acc_sh):
  c = lax.axis_index("c")
  s = lax.axis_index("s")
  base = (c * NS + s) * EPT

  _zero_vmem_1d(z_v, STR)
  pltpu.sync_copy(z_v, acc_sh.at[pl.ds(s * STR, STR)])
  plsc.subcore_barrier()

  def chunk(i, _):
    off = base + i * CH
    pltpu.sync_copy(dst_hbm.at[pl.ds(off, CH)], idx_v)
    pltpu.sync_copy(ew_hbm.at[pl.ds(off, CH)], ew_v)
    pltpu.sync_copy(ew_v, acc_sh.at[idx_v], add=True)
    return 0
  lax.fori_loop(0, NCHUNK, chunk, 0)

  plsc.subcore_barrier()
  pltpu.sync_copy(acc_sh.at[pl.ds(s * STR, STR)],
                  out_hbm.at[pl.ds(c * APAD + s * STR, STR)])


_deg_call = pl.kernel(
    _deg_body,
    out_type=jax.ShapeDtypeStruct((NC * APAD,), jnp.float32),
    mesh=plsc.VectorSubcoreMesh(core_axis_name="c", subcore_axis_name="s"),
    scratch_types=[
        pltpu.VMEM((CH,), jnp.int32),
        pltpu.VMEM((CH,), jnp.float32),
        pltpu.VMEM((STR,), jnp.float32),
        pltpu.VMEM_SHARED((APAD,), jnp.float32),
    ],
)


def _agg_body(hp_hbm, src_hbm, dst_hbm, ew_hbm, out_hbm,
              sidx, didx, ewv, rows, zrows, acc_sh, gsem):
  c = lax.axis_index("c")
  s = lax.axis_index("s")
  base = (c * NS + s) * EPT

  _zero_vmem_rows(zrows, 128)
  for t in range(5):
    pltpu.sync_copy(zrows, acc_sh.at[pl.ds(s * STR + t * 128, 128)])
  plsc.subcore_barrier()

  def chunk(i, _):
    off = base + i * CH
    pltpu.sync_copy(src_hbm.at[pl.ds(off, CH)], sidx)
    pltpu.sync_copy(ew_hbm.at[pl.ds(off, CH)], ewv)
    pltpu.sync_copy(dst_hbm.at[pl.ds(off, CH)], didx)
    pltpu.async_copy(hp_hbm.at[sidx], rows, gsem).wait()

    def grp(g, _):
      wv = ewv[pl.ds(g * 16, 16)]
      for e in range(16):
        w = _bcast16(wv, e)
        r = g * 16 + e
        for j in range(D // 16):
          rows[r, pl.ds(j * 16, 16)] = rows[r, pl.ds(j * 16, 16)] * w
      return 0
    lax.fori_loop(0, CH // 16, grp, 0)

    pltpu.sync_copy(rows, acc_sh.at[didx], add=True)
    return 0
  lax.fori_loop(0, NCHUNK, chunk, 0)

  plsc.subcore_barrier()
  pltpu.sync_copy(acc_sh.at[pl.ds(s * STR, STR)],
                  out_hbm.at[c, pl.ds(s * STR, STR)])


_agg_call = pl.kernel(
    _agg_body,
    out_type=jax.ShapeDtypeStruct((NC, APAD, D), jnp.float32),
    mesh=plsc.VectorSubcoreMesh(core_axis_name="c", subcore_axis_name="s"),
    scratch_types=[
        pltpu.VMEM((CH,), jnp.int32),
        pltpu.VMEM((CH,), jnp.int32),
        pltpu.VMEM((CH,), jnp.float32),
        pltpu.VMEM((CH, D), jnp.float32),
        pltpu.VMEM((128, D), jnp.float32),
        pltpu.VMEM_SHARED((APAD, D), jnp.float32),
        pltpu.SemaphoreType.DMA,
    ],
)


def _leaky(x):
  return jnp.where(x >= 0.0, x, 0.1 * x)


def _bn(x, gamma, beta):
  mu = jnp.mean(x, axis=0, keepdims=True)
  xc = x - mu
  var = jnp.mean(xc * xc, axis=0, keepdims=True)
  return gamma * xc * lax.rsqrt(var + 1e-5) + beta


def _tc1_body(x_ref, d0_ref, d1_ref, g_ref, bt_ref, w_ref, h1p_ref, dinv_ref):
  x = x_ref[...]
  xa = _leaky(_bn(x, g_ref[...], bt_ref[...]))
  deg = d0_ref[...] + d1_ref[...] + 1.0
  dinv = jnp.where(deg > 0.0, lax.rsqrt(deg), 0.0)
  h = jnp.dot(xa, w_ref[...], preferred_element_type=jnp.float32)
  h1p_ref[...] = dinv * h
  dinv_ref[...] = dinv


def _tc2_body(p0_ref, p1_ref, hp_ref, dinv_ref, b1_ref, g_ref, bt_ref, w_ref,
              h2p_ref):
  dinv = dinv_ref[...]
  o1 = dinv * (p0_ref[...] + p1_ref[...] + hp_ref[...]) + b1_ref[...]
  a = _leaky(_bn(_leaky(o1), g_ref[...], bt_ref[...]))
  h2 = jnp.dot(a, w_ref[...], preferred_element_type=jnp.float32)
  h2p_ref[...] = dinv * h2


def _tc3_body(q0_ref, q1_ref, hp_ref, dinv_ref, b2_ref, out_ref):
  o = dinv_ref[...] * (q0_ref[...] + q1_ref[...] + hp_ref[...]) + b2_ref[...]
  out_ref[...] = _leaky(o)


_f32 = jnp.float32
_tc1_call = pl.pallas_call(
    _tc1_body,
    out_shape=(jax.ShapeDtypeStruct((N, D), _f32),
               jax.ShapeDtypeStruct((N, 1), _f32)),
)
_tc2_call = pl.pallas_call(
    _tc2_body,
    out_shape=jax.ShapeDtypeStruct((N, D), _f32),
)
_tc3_call = pl.pallas_call(
    _tc3_body,
    out_shape=jax.ShapeDtypeStruct((N, D), _f32),
)


def kernel(x, edge_index, edge_attr, bn1_gamma, bn1_beta, W1, b1,
           bn2_gamma, bn2_beta, W2, b2):
  src = edge_index[0]
  dst = edge_index[1]
  ew = edge_attr[:, 0]

  deg_parts = _deg_call(dst, ew)
  d0 = deg_parts[:N, None]
  d1 = deg_parts[APAD:APAD + N, None]

  h1p, dinv = _tc1_call(x, d0, d1, bn1_gamma[None, :], bn1_beta[None, :], W1)

  p = _agg_call(h1p, src, dst, ew)
  h2p = _tc2_call(p[0, :N], p[1, :N], h1p, dinv, b1[None, :],
                  bn2_gamma[None, :], bn2_beta[None, :], W2)

  q = _agg_call(h2p, src, dst, ew)
  out = _tc3_call(q[0, :N], q[1, :N], h2p, dinv, b2[None, :])
  return (out, edge_index)
